# Initial kernel scaffold; baseline (speedup 1.0000x reference)
import jax
import jax.numpy as jnp
from jax.experimental import pallas as pl


def kernel(node_input, edge_src, edge_dst, edge_attr, edge_scalar_attr, W_self, W_fc1, W_fc2, W_tp, W_out):
    raise NotImplementedError("write your pallas kernel here")


# speedup vs baseline: 3.7108x; 3.7108x over previous
import functools
import math

import jax
import jax.numpy as jnp
from jax import lax
from jax.experimental import pallas as pl
from jax.experimental.pallas import tpu as pltpu
from jax.experimental.pallas import tpu_sc as plsc

_N = 10000
_E = 320000
_D_IN = 128
_D_EDGE = 4
_D_SCALAR = 16
_FC0 = 64
_FC1 = 64
_NUM_NEIGHBORS = 32.0
_MIX = math.pi / 8.0

_NC = 2
_NS = 16
_NW = _NC * _NS
_B = 80
_EPW = _E // _NW
_NB = _EPW // _B
_RPT = 632
_NPAD = _RPT * _NS

_NODE_BLK = 2000
_EDGE_BLK = 3200


def _node_linear_body(x_ref, w_ref, nf_ref, so_ref):
    t = jnp.dot(x_ref[...], w_ref[...], preferred_element_type=jnp.float32)
    nf_ref[...] = t[:, :_D_IN]
    so_ref[...] = t[:, _D_IN:]


def _edge_mlp_body(esa_ref, ea_ref, w1_ref, w2_ref, wtp_ref, m_ref):
    h = jax.nn.gelu(
        jnp.dot(esa_ref[...], w1_ref[...], preferred_element_type=jnp.float32))
    h = jax.nn.gelu(
        jnp.dot(h, w2_ref[...], preferred_element_type=jnp.float32))
    ea = ea_ref[...]
    p = jnp.concatenate([h * ea[:, v:v + 1] for v in range(_D_EDGE)], axis=1)
    m_ref[...] = jnp.dot(p, wtp_ref[...], preferred_element_type=jnp.float32)


def _final_body(agg_ref, so_ref, wout_ref, o_ref):
    a = agg_ref[0] + agg_ref[1]
    o_ref[...] = so_ref[...] * math.cos(_MIX) + jnp.dot(
        a, wout_ref[...], preferred_element_type=jnp.float32)


def _sc_edge_body(nf_hbm, m_hbm, src_hbm, dst_hbm, zeros_hbm, out_hbm,
                  src_v, dst_v, rows_v, m_v, agg_sh, sem):
    cid = lax.axis_index("c")
    sid = lax.axis_index("s")
    wid = sid * _NC + cid

    pltpu.sync_copy(zeros_hbm.at[pl.ds(sid * _RPT, _RPT)],
                    agg_sh.at[pl.ds(sid * _RPT, _RPT)])
    plsc.subcore_barrier()

    base = wid * _EPW

    def batch_body(i, carry):
        start = base + i * _B
        pltpu.sync_copy(src_hbm.at[pl.ds(start, _B)], src_v)
        pltpu.sync_copy(dst_hbm.at[pl.ds(start, _B)], dst_v)
        pltpu.async_copy(nf_hbm.at[src_v], rows_v, sem).wait()
        pltpu.sync_copy(m_hbm.at[pl.ds(start, _B)], m_v)

        def row_body(r, c2):
            for c in range(_D_IN // 16):
                sl = pl.ds(c * 16, 16)
                rows_v[r, sl] = rows_v[r, sl] * m_v[r, sl]
            return c2

        lax.fori_loop(0, _B, row_body, 0)
        pltpu.sync_copy(rows_v, agg_sh.at[dst_v], add=True)
        return carry

    lax.fori_loop(0, _NB, batch_body, 0)
    plsc.subcore_barrier()

    pltpu.sync_copy(agg_sh.at[pl.ds(sid * _RPT, _RPT)],
                    out_hbm.at[cid, pl.ds(sid * _RPT, _RPT)])


def _sc_edge_call(nf, m, src, dst, zeros):
    mesh = plsc.VectorSubcoreMesh(core_axis_name="c", subcore_axis_name="s")
    f = functools.partial(
        pl.kernel,
        out_type=jax.ShapeDtypeStruct((_NC, _NPAD, _D_IN), jnp.float32),
        mesh=mesh,
        scratch_types=[
            pltpu.VMEM((_B,), jnp.int32),
            pltpu.VMEM((_B,), jnp.int32),
            pltpu.VMEM((_B, _D_IN), jnp.float32),
            pltpu.VMEM((_B, _D_IN), jnp.float32),
            pltpu.VMEM_SHARED((_NPAD, _D_IN), jnp.float32),
            pltpu.SemaphoreType.DMA,
        ],
    )(_sc_edge_body)
    return f(nf, m, src, dst, zeros)


def kernel(node_input, edge_src, edge_dst, edge_attr, edge_scalar_attr,
           W_self, W_fc1, W_fc2, W_tp, W_out):
    w_self = W_self / math.sqrt(_D_IN)
    w1 = W_fc1 / math.sqrt(_D_SCALAR)
    w2 = W_fc2 / math.sqrt(_FC0)
    wtp = (jnp.transpose(W_tp, (2, 0, 1)).reshape(_D_EDGE * _FC1, _D_IN)
           / (math.sqrt(_FC1) * math.sqrt(_D_EDGE)))
    wout = W_out * (math.sin(_MIX) / (math.sqrt(_NUM_NEIGHBORS)
                                      * math.sqrt(_D_IN)))

    nf, node_self_out = pl.pallas_call(
        _node_linear_body,
        grid=(_N // _NODE_BLK,),
        in_specs=[
            pl.BlockSpec((_NODE_BLK, _D_IN), lambda i: (i, 0)),
            pl.BlockSpec((_D_IN, 2 * _D_IN), lambda i: (0, 0)),
        ],
        out_specs=[
            pl.BlockSpec((_NODE_BLK, _D_IN), lambda i: (i, 0)),
            pl.BlockSpec((_NODE_BLK, _D_IN), lambda i: (i, 0)),
        ],
        out_shape=[
            jax.ShapeDtypeStruct((_N, _D_IN), jnp.float32),
            jax.ShapeDtypeStruct((_N, _D_IN), jnp.float32),
        ],
    )(node_input, w_self)

    m = pl.pallas_call(
        _edge_mlp_body,
        grid=(_E // _EDGE_BLK,),
        in_specs=[
            pl.BlockSpec((_EDGE_BLK, _D_SCALAR), lambda i: (i, 0)),
            pl.BlockSpec((_EDGE_BLK, _D_EDGE), lambda i: (i, 0)),
            pl.BlockSpec((_D_SCALAR, _FC0), lambda i: (0, 0)),
            pl.BlockSpec((_FC0, _FC1), lambda i: (0, 0)),
            pl.BlockSpec((_D_EDGE * _FC1, _D_IN), lambda i: (0, 0)),
        ],
        out_specs=pl.BlockSpec((_EDGE_BLK, _D_IN), lambda i: (i, 0)),
        out_shape=jax.ShapeDtypeStruct((_E, _D_IN), jnp.float32),
    )(edge_scalar_attr, edge_attr, w1, w2, wtp)

    zeros = jnp.zeros((_NPAD, _D_IN), jnp.float32)
    agg2 = _sc_edge_call(nf, m, edge_src, edge_dst, zeros)

    out = pl.pallas_call(
        _final_body,
        grid=(_N // _NODE_BLK,),
        in_specs=[
            pl.BlockSpec((_NC, _NODE_BLK, _D_IN), lambda i: (0, i, 0)),
            pl.BlockSpec((_NODE_BLK, _D_IN), lambda i: (i, 0)),
            pl.BlockSpec((_D_IN, _D_IN), lambda i: (0, 0)),
        ],
        out_specs=pl.BlockSpec((_NODE_BLK, _D_IN), lambda i: (i, 0)),
        out_shape=jax.ShapeDtypeStruct((_N, _D_IN), jnp.float32),
    )(agg2, node_self_out, wout)
    return out


# speedup vs baseline: 5.4434x; 1.4669x over previous
import functools
import math

import jax
import jax.numpy as jnp
from jax import lax
from jax.experimental import pallas as pl
from jax.experimental.pallas import tpu as pltpu
from jax.experimental.pallas import tpu_sc as plsc

_N = 10000
_E = 320000
_D_IN = 128
_D_EDGE = 4
_D_SCALAR = 16
_FC0 = 64
_FC1 = 64
_NUM_NEIGHBORS = 32.0
_MIX = math.pi / 8.0

_NC = 2
_NS = 16
_NW = _NC * _NS
_B = 40
_EPW = _E // _NW
_NB = _EPW // _B
_RPT_BIG = 632
_RPT_SMALL = 624

_NODE_BLK = 2000
_EDGE_BLK = 3200


def _node_linear_body(x_ref, w_ref, nf_ref, so_ref):
    t = jnp.dot(x_ref[...], w_ref[...], preferred_element_type=jnp.float32)
    nf_ref[...] = t[:, :_D_IN]
    so_ref[...] = t[:, _D_IN:]


def _edge_mlp_body(esa_ref, ea_ref, w1_ref, w2_ref, wtp_ref, m_ref):
    h = jax.nn.gelu(
        jnp.dot(esa_ref[...], w1_ref[...], preferred_element_type=jnp.float32))
    h = jax.nn.gelu(
        jnp.dot(h, w2_ref[...], preferred_element_type=jnp.float32))
    ea = ea_ref[...]
    p = jnp.concatenate([h * ea[:, v:v + 1] for v in range(_D_EDGE)], axis=1)
    m_ref[...] = jnp.dot(p, wtp_ref[...], preferred_element_type=jnp.float32)


def _final_body(agg_ref, so_ref, wout_ref, o_ref):
    a = agg_ref[0] + agg_ref[1]
    o_ref[...] = so_ref[...] * math.cos(_MIX) + jnp.dot(
        a, wout_ref[...], preferred_element_type=jnp.float32)


def _sc_edge_body(nf_hbm, m_hbm, src_hbm, dst_hbm, zeros_hbm, out_hbm,
                  src_all, dst_v, rows_v, m_v, agg_sh, gsem, msem, ssem,
                  dsem):
    cid = lax.axis_index("c")
    sid = lax.axis_index("s")
    wid = sid * _NC + cid
    base = wid * _EPW

    pltpu.sync_copy(src_hbm.at[wid], src_all)

    @pl.when(sid < 2)
    def _():
        off = sid * _RPT_BIG
        pltpu.sync_copy(zeros_hbm.at[pl.ds(off, _RPT_BIG)],
                        agg_sh.at[pl.ds(off, _RPT_BIG)])

    @pl.when(sid >= 2)
    def _():
        off = sid * _RPT_SMALL + 2 * (_RPT_BIG - _RPT_SMALL)
        pltpu.sync_copy(zeros_hbm.at[pl.ds(off, _RPT_SMALL)],
                        agg_sh.at[pl.ds(off, _RPT_SMALL)])

    plsc.subcore_barrier()

    def issue(i, slot):
        pltpu.async_copy(nf_hbm.at[src_all.at[pl.ds(i * _B, _B)]],
                         rows_v[slot], gsem[slot])
        pltpu.async_copy(m_hbm.at[pl.ds(base + i * _B, _B)], m_v[slot],
                         msem[slot])
        pltpu.async_copy(dst_hbm.at[wid, i], dst_v[slot], dsem[slot])

    def process(i, slot):
        pltpu.make_async_copy(nf_hbm.at[pl.ds(0, _B)], rows_v[slot],
                              gsem[slot]).wait()
        pltpu.make_async_copy(m_hbm.at[pl.ds(base, _B)], m_v[slot],
                              msem[slot]).wait()

        @plsc.parallel_loop(0, _B, 1, unroll=2)
        def _mul(r):
            for c in range(_D_IN // 16):
                sl = pl.ds(c * 16, 16)
                rows_v[slot][r, sl] = rows_v[slot][r, sl] * m_v[slot][r, sl]

        pltpu.make_async_copy(dst_hbm.at[wid, 0], dst_v[slot],
                              dsem[slot]).wait()
        pltpu.async_copy(rows_v[slot], agg_sh.at[dst_v[slot]],
                         ssem[slot], add=True)

    def wait_scatter(slot):
        pltpu.make_async_copy(rows_v[slot], agg_sh.at[dst_v[slot]],
                              ssem[slot]).wait()

    issue(0, 0)
    issue(1, 1)
    process(0, 0)
    issue(2, 2)
    process(1, 1)

    def triple(k, carry):
        for j in range(3):
            i = 2 + k * 3 + j
            wait_scatter(j)
            issue(i + 1, j)
            process(i, (2 + j) % 3)
        return carry

    lax.fori_loop(0, (_NB - 4) // 3, triple, 0)

    wait_scatter(0)
    issue(_NB - 1, 0)
    process(_NB - 2, 2)
    wait_scatter(1)
    process(_NB - 1, 0)
    wait_scatter(2)
    wait_scatter(0)
    plsc.subcore_barrier()

    @pl.when(sid < 2)
    def _():
        off = sid * _RPT_BIG
        pltpu.sync_copy(agg_sh.at[pl.ds(off, _RPT_BIG)],
                        out_hbm.at[cid, pl.ds(off, _RPT_BIG)])

    @pl.when(sid >= 2)
    def _():
        off = sid * _RPT_SMALL + 2 * (_RPT_BIG - _RPT_SMALL)
        pltpu.sync_copy(agg_sh.at[pl.ds(off, _RPT_SMALL)],
                        out_hbm.at[cid, pl.ds(off, _RPT_SMALL)])


def _sc_edge_call(nf, m, src, dst, zeros):
    mesh = plsc.VectorSubcoreMesh(core_axis_name="c", subcore_axis_name="s")
    f = functools.partial(
        pl.kernel,
        out_type=jax.ShapeDtypeStruct((_NC, _N, _D_IN), jnp.float32),
        mesh=mesh,
        scratch_types=[
            pltpu.VMEM((_EPW,), jnp.int32),
            [pltpu.VMEM((_B,), jnp.int32) for _ in range(3)],
            [pltpu.VMEM((_B, _D_IN), jnp.float32) for _ in range(3)],
            [pltpu.VMEM((_B, _D_IN), jnp.float32) for _ in range(3)],
            pltpu.VMEM_SHARED((_N, _D_IN), jnp.float32),
            [pltpu.SemaphoreType.DMA for _ in range(3)],
            [pltpu.SemaphoreType.DMA for _ in range(3)],
            [pltpu.SemaphoreType.DMA for _ in range(3)],
            [pltpu.SemaphoreType.DMA for _ in range(3)],
        ],
    )(_sc_edge_body)
    return f(nf, m, src.reshape(_NW, _EPW), dst.reshape(_NW, _NB, _B),
             zeros)


def kernel(node_input, edge_src, edge_dst, edge_attr, edge_scalar_attr,
           W_self, W_fc1, W_fc2, W_tp, W_out):
    w_self = W_self / math.sqrt(_D_IN)
    w1 = W_fc1 / math.sqrt(_D_SCALAR)
    w2 = W_fc2 / math.sqrt(_FC0)
    wtp = (jnp.transpose(W_tp, (2, 0, 1)).reshape(_D_EDGE * _FC1, _D_IN)
           / (math.sqrt(_FC1) * math.sqrt(_D_EDGE)))
    wout = W_out * (math.sin(_MIX) / (math.sqrt(_NUM_NEIGHBORS)
                                      * math.sqrt(_D_IN)))

    nf, node_self_out = pl.pallas_call(
        _node_linear_body,
        grid=(_N // _NODE_BLK,),
        in_specs=[
            pl.BlockSpec((_NODE_BLK, _D_IN), lambda i: (i, 0)),
            pl.BlockSpec((_D_IN, 2 * _D_IN), lambda i: (0, 0)),
        ],
        out_specs=[
            pl.BlockSpec((_NODE_BLK, _D_IN), lambda i: (i, 0)),
            pl.BlockSpec((_NODE_BLK, _D_IN), lambda i: (i, 0)),
        ],
        out_shape=[
            jax.ShapeDtypeStruct((_N, _D_IN), jnp.float32),
            jax.ShapeDtypeStruct((_N, _D_IN), jnp.float32),
        ],
    )(node_input, w_self)

    m = pl.pallas_call(
        _edge_mlp_body,
        grid=(_E // _EDGE_BLK,),
        in_specs=[
            pl.BlockSpec((_EDGE_BLK, _D_SCALAR), lambda i: (i, 0)),
            pl.BlockSpec((_EDGE_BLK, _D_EDGE), lambda i: (i, 0)),
            pl.BlockSpec((_D_SCALAR, _FC0), lambda i: (0, 0)),
            pl.BlockSpec((_FC0, _FC1), lambda i: (0, 0)),
            pl.BlockSpec((_D_EDGE * _FC1, _D_IN), lambda i: (0, 0)),
        ],
        out_specs=pl.BlockSpec((_EDGE_BLK, _D_IN), lambda i: (i, 0)),
        out_shape=jax.ShapeDtypeStruct((_E, _D_IN), jnp.float32),
    )(edge_scalar_attr, edge_attr, w1, w2, wtp)

    zeros = jnp.zeros((_N, _D_IN), jnp.float32)
    agg2 = _sc_edge_call(nf, m, edge_src, edge_dst, zeros)

    out = pl.pallas_call(
        _final_body,
        grid=(_N // _NODE_BLK,),
        in_specs=[
            pl.BlockSpec((_NC, _NODE_BLK, _D_IN), lambda i: (0, i, 0)),
            pl.BlockSpec((_NODE_BLK, _D_IN), lambda i: (i, 0)),
            pl.BlockSpec((_D_IN, _D_IN), lambda i: (0, 0)),
        ],
        out_specs=pl.BlockSpec((_NODE_BLK, _D_IN), lambda i: (i, 0)),
        out_shape=jax.ShapeDtypeStruct((_N, _D_IN), jnp.float32),
    )(agg2, node_self_out, wout)
    return out


# speedup vs baseline: 9.6630x; 1.7752x over previous
import functools
import math

import jax
import jax.numpy as jnp
from jax import lax
from jax.experimental import pallas as pl
from jax.experimental.pallas import tpu as pltpu
from jax.experimental.pallas import tpu_sc as plsc

_N = 10000
_E = 320000
_D_IN = 128
_D_EDGE = 4
_D_SCALAR = 16
_FC0 = 64
_FC1 = 64
_NUM_NEIGHBORS = 32.0
_MIX = math.pi / 8.0

_NC = 2
_NS = 16
_NW = _NC * _NS
_B = 40
_EPW = _E // _NW
_NB = _EPW // _B
_RPT_BIG = 632
_RPT_SMALL = 624

_NODE_BLK = 2000
_EDGE_BLK = 3200


def _node_linear_body(x_ref, w_ref, nf_ref, so_ref):
    t = jnp.dot(x_ref[...], w_ref[...], preferred_element_type=jnp.float32)
    nf_ref[...] = t[:, :_D_IN]
    so_ref[...] = t[:, _D_IN:]


def _edge_mlp_body(esa_ref, ea_ref, w1t_ref, w2t_ref, wtpt_ref, m_ref):
    h = jax.nn.gelu(
        jnp.dot(w1t_ref[...], esa_ref[...], preferred_element_type=jnp.float32))
    h = jax.nn.gelu(
        jnp.dot(w2t_ref[...], h, preferred_element_type=jnp.float32))
    hb = h.astype(jnp.bfloat16)
    ea = ea_ref[...].astype(jnp.bfloat16)
    p = jnp.concatenate([hb * ea[v:v + 1, :] for v in range(_D_EDGE)], axis=0)
    mt = jnp.dot(wtpt_ref[...], p, preferred_element_type=jnp.float32)
    m_ref[...] = mt.T


def _final_body(agg_ref, so_ref, wout_ref, o_ref):
    a = agg_ref[0] + agg_ref[1]
    o_ref[...] = so_ref[...] * math.cos(_MIX) + jnp.dot(
        a, wout_ref[...], preferred_element_type=jnp.float32)


def _sc_edge_body(nf_hbm, m_hbm, src_hbm, dst_hbm, zeros_hbm, out_hbm,
                  src_all, dst_v, rows_v, m_v, agg_sh, gsem, msem, ssem,
                  dsem):
    cid = lax.axis_index("c")
    sid = lax.axis_index("s")
    wid = sid * _NC + cid
    base = wid * _EPW

    pltpu.sync_copy(src_hbm.at[wid], src_all)

    @pl.when(sid < 2)
    def _():
        off = sid * _RPT_BIG
        pltpu.sync_copy(zeros_hbm.at[pl.ds(off, _RPT_BIG)],
                        agg_sh.at[pl.ds(off, _RPT_BIG)])

    @pl.when(sid >= 2)
    def _():
        off = sid * _RPT_SMALL + 2 * (_RPT_BIG - _RPT_SMALL)
        pltpu.sync_copy(zeros_hbm.at[pl.ds(off, _RPT_SMALL)],
                        agg_sh.at[pl.ds(off, _RPT_SMALL)])

    plsc.subcore_barrier()

    def issue(i, slot):
        pltpu.async_copy(nf_hbm.at[src_all.at[pl.ds(i * _B, _B)]],
                         rows_v[slot], gsem[slot])
        pltpu.async_copy(m_hbm.at[pl.ds(base + i * _B, _B)], m_v[slot],
                         msem[slot])
        pltpu.async_copy(dst_hbm.at[wid, i], dst_v[slot], dsem[slot])

    def process(i, slot):
        pltpu.make_async_copy(nf_hbm.at[pl.ds(0, _B)], rows_v[slot],
                              gsem[slot]).wait()
        pltpu.make_async_copy(m_hbm.at[pl.ds(base, _B)], m_v[slot],
                              msem[slot]).wait()

        @plsc.parallel_loop(0, _B, 1, unroll=2)
        def _mul(r):
            for c in range(_D_IN // 16):
                sl = pl.ds(c * 16, 16)
                rows_v[slot][r, sl] = rows_v[slot][r, sl] * m_v[slot][r, sl]

        pltpu.make_async_copy(dst_hbm.at[wid, 0], dst_v[slot],
                              dsem[slot]).wait()
        pltpu.async_copy(rows_v[slot], agg_sh.at[dst_v[slot]],
                         ssem[slot], add=True)

    def wait_scatter(slot):
        pltpu.make_async_copy(rows_v[slot], agg_sh.at[dst_v[slot]],
                              ssem[slot]).wait()

    issue(0, 0)
    issue(1, 1)
    process(0, 0)
    issue(2, 2)
    process(1, 1)

    def triple(k, carry):
        for j in range(3):
            i = 2 + k * 3 + j
            wait_scatter(j)
            issue(i + 1, j)
            process(i, (2 + j) % 3)
        return carry

    lax.fori_loop(0, (_NB - 4) // 3, triple, 0)

    wait_scatter(0)
    issue(_NB - 1, 0)
    process(_NB - 2, 2)
    wait_scatter(1)
    process(_NB - 1, 0)
    wait_scatter(2)
    wait_scatter(0)
    plsc.subcore_barrier()

    @pl.when(sid < 2)
    def _():
        off = sid * _RPT_BIG
        pltpu.sync_copy(agg_sh.at[pl.ds(off, _RPT_BIG)],
                        out_hbm.at[cid, pl.ds(off, _RPT_BIG)])

    @pl.when(sid >= 2)
    def _():
        off = sid * _RPT_SMALL + 2 * (_RPT_BIG - _RPT_SMALL)
        pltpu.sync_copy(agg_sh.at[pl.ds(off, _RPT_SMALL)],
                        out_hbm.at[cid, pl.ds(off, _RPT_SMALL)])


def _sc_edge_call(nf, m, src, dst, zeros):
    mesh = plsc.VectorSubcoreMesh(core_axis_name="c", subcore_axis_name="s")
    f = functools.partial(
        pl.kernel,
        out_type=jax.ShapeDtypeStruct((_NC, _N, _D_IN), jnp.float32),
        mesh=mesh,
        scratch_types=[
            pltpu.VMEM((_EPW,), jnp.int32),
            [pltpu.VMEM((_B,), jnp.int32) for _ in range(3)],
            [pltpu.VMEM((_B, _D_IN), jnp.float32) for _ in range(3)],
            [pltpu.VMEM((_B, _D_IN), jnp.float32) for _ in range(3)],
            pltpu.VMEM_SHARED((_N, _D_IN), jnp.float32),
            [pltpu.SemaphoreType.DMA for _ in range(3)],
            [pltpu.SemaphoreType.DMA for _ in range(3)],
            [pltpu.SemaphoreType.DMA for _ in range(3)],
            [pltpu.SemaphoreType.DMA for _ in range(3)],
        ],
    )(_sc_edge_body)
    return f(nf, m, src.reshape(_NW, _EPW), dst.reshape(_NW, _NB, _B),
             zeros)


def kernel(node_input, edge_src, edge_dst, edge_attr, edge_scalar_attr,
           W_self, W_fc1, W_fc2, W_tp, W_out):
    w_self = W_self / math.sqrt(_D_IN)
    w1t = (W_fc1 / math.sqrt(_D_SCALAR)).T
    w2t = (W_fc2 / math.sqrt(_FC0)).T
    wtpt = (jnp.transpose(W_tp, (1, 2, 0)).reshape(_D_IN, _D_EDGE * _FC1)
            / (math.sqrt(_FC1) * math.sqrt(_D_EDGE))).astype(jnp.bfloat16)
    wout = W_out * (math.sin(_MIX) / (math.sqrt(_NUM_NEIGHBORS)
                                      * math.sqrt(_D_IN)))

    nf, node_self_out = pl.pallas_call(
        _node_linear_body,
        grid=(_N // _NODE_BLK,),
        in_specs=[
            pl.BlockSpec((_NODE_BLK, _D_IN), lambda i: (i, 0)),
            pl.BlockSpec((_D_IN, 2 * _D_IN), lambda i: (0, 0)),
        ],
        out_specs=[
            pl.BlockSpec((_NODE_BLK, _D_IN), lambda i: (i, 0)),
            pl.BlockSpec((_NODE_BLK, _D_IN), lambda i: (i, 0)),
        ],
        out_shape=[
            jax.ShapeDtypeStruct((_N, _D_IN), jnp.float32),
            jax.ShapeDtypeStruct((_N, _D_IN), jnp.float32),
        ],
    )(node_input, w_self)

    m = pl.pallas_call(
        _edge_mlp_body,
        grid=(_E // _EDGE_BLK,),
        in_specs=[
            pl.BlockSpec((_D_SCALAR, _EDGE_BLK), lambda i: (0, i)),
            pl.BlockSpec((_D_EDGE, _EDGE_BLK), lambda i: (0, i)),
            pl.BlockSpec((_FC0, _D_SCALAR), lambda i: (0, 0)),
            pl.BlockSpec((_FC1, _FC0), lambda i: (0, 0)),
            pl.BlockSpec((_D_IN, _D_EDGE * _FC1), lambda i: (0, 0)),
        ],
        out_specs=pl.BlockSpec((_EDGE_BLK, _D_IN), lambda i: (i, 0)),
        out_shape=jax.ShapeDtypeStruct((_E, _D_IN), jnp.float32),
    )(edge_scalar_attr.T, edge_attr.T, w1t, w2t, wtpt)

    zeros = jnp.zeros((_N, _D_IN), jnp.float32)
    agg2 = _sc_edge_call(nf, m, edge_src, edge_dst, zeros)

    out = pl.pallas_call(
        _final_body,
        grid=(_N // _NODE_BLK,),
        in_specs=[
            pl.BlockSpec((_NC, _NODE_BLK, _D_IN), lambda i: (0, i, 0)),
            pl.BlockSpec((_NODE_BLK, _D_IN), lambda i: (i, 0)),
            pl.BlockSpec((_D_IN, _D_IN), lambda i: (0, 0)),
        ],
        out_specs=pl.BlockSpec((_NODE_BLK, _D_IN), lambda i: (i, 0)),
        out_shape=jax.ShapeDtypeStruct((_N, _D_IN), jnp.float32),
    )(agg2, node_self_out, wout)
    return out


# speedup vs baseline: 10.5933x; 1.0963x over previous
import functools
import math

import jax
import jax.numpy as jnp
from jax import lax
from jax.experimental import pallas as pl
from jax.experimental.pallas import tpu as pltpu
from jax.experimental.pallas import tpu_sc as plsc

_N = 10000
_E = 320000
_D_IN = 128
_D_EDGE = 4
_D_SCALAR = 16
_FC0 = 64
_FC1 = 64
_NUM_NEIGHBORS = 32.0
_MIX = math.pi / 8.0

_NC = 2
_NS = 16
_NW = _NC * _NS
_B = 40
_EPW = _E // _NW
_NB = _EPW // _B
_RPT_BIG = 632
_RPT_SMALL = 624

_NODE_BLK = 2000
_EDGE_BLK = 3200


def _node_linear_body(x_ref, w_ref, nf_ref, so_ref):
    t = jnp.dot(x_ref[...], w_ref[...], preferred_element_type=jnp.float32)
    nf_ref[...] = t[:, :_D_IN]
    so_ref[...] = t[:, _D_IN:]


def _edge_mlp_body(esa_ref, ea_ref, w1t_ref, w2t_ref, wtpt_ref, m_ref):
    h = jax.nn.gelu(
        jnp.dot(w1t_ref[...], esa_ref[...], preferred_element_type=jnp.float32))
    h = jax.nn.gelu(
        jnp.dot(w2t_ref[...], h, preferred_element_type=jnp.float32))
    hb = h.astype(jnp.bfloat16)
    ea = ea_ref[...].astype(jnp.bfloat16)
    p = jnp.concatenate([hb * ea[v:v + 1, :] for v in range(_D_EDGE)], axis=0)
    mt = jnp.dot(wtpt_ref[...], p, preferred_element_type=jnp.float32)
    m_ref[...] = mt.T


def _final_body(agg_a_ref, agg_b_ref, so_ref, wout_ref, o_ref):
    a = (agg_a_ref[0] + agg_a_ref[1]) + (agg_b_ref[0] + agg_b_ref[1])
    o_ref[...] = so_ref[...] * math.cos(_MIX) + jnp.dot(
        a, wout_ref[...], preferred_element_type=jnp.float32)


def _sc_edge_body(epw, nf_hbm, m_hbm, src_hbm, dst_hbm, zeros_hbm, out_hbm,
                  src_all, dst_v, rows_v, m_v, agg_sh, gsem, msem, ssem,
                  dsem):
    nb = epw // _B
    cid = lax.axis_index("c")
    sid = lax.axis_index("s")
    wid = sid * _NC + cid
    base = wid * epw

    pltpu.sync_copy(src_hbm.at[wid], src_all)

    @pl.when(sid < 2)
    def _():
        off = sid * _RPT_BIG
        pltpu.sync_copy(zeros_hbm.at[pl.ds(off, _RPT_BIG)],
                        agg_sh.at[pl.ds(off, _RPT_BIG)])

    @pl.when(sid >= 2)
    def _():
        off = sid * _RPT_SMALL + 2 * (_RPT_BIG - _RPT_SMALL)
        pltpu.sync_copy(zeros_hbm.at[pl.ds(off, _RPT_SMALL)],
                        agg_sh.at[pl.ds(off, _RPT_SMALL)])

    plsc.subcore_barrier()

    def issue(i, slot):
        pltpu.async_copy(nf_hbm.at[src_all.at[pl.ds(i * _B, _B)]],
                         rows_v[slot], gsem[slot])
        pltpu.async_copy(m_hbm.at[pl.ds(base + i * _B, _B)], m_v[slot],
                         msem[slot])
        pltpu.async_copy(dst_hbm.at[pl.ds(base + i * _B, _B)], dst_v[slot],
                         dsem[slot])

    def process(i, slot):
        pltpu.make_async_copy(nf_hbm.at[pl.ds(0, _B)], rows_v[slot],
                              gsem[slot]).wait()
        pltpu.make_async_copy(m_hbm.at[pl.ds(base, _B)], m_v[slot],
                              msem[slot]).wait()

        @plsc.parallel_loop(0, _B, 1, unroll=2)
        def _mul(r):
            for c in range(_D_IN // 16):
                sl = pl.ds(c * 16, 16)
                rows_v[slot][r, sl] = rows_v[slot][r, sl] * m_v[slot][r, sl]

        pltpu.make_async_copy(dst_hbm.at[pl.ds(base, _B)], dst_v[slot],
                              dsem[slot]).wait()
        pltpu.async_copy(rows_v[slot], agg_sh.at[dst_v[slot]],
                         ssem[slot], add=True)

    def wait_scatter(slot):
        pltpu.make_async_copy(rows_v[slot], agg_sh.at[dst_v[slot]],
                              ssem[slot]).wait()

    tail = 2 + (nb - 4) % 3
    loops = (nb - 2 - tail) // 3
    issue(0, 0)
    issue(1, 1)
    process(0, 0)
    issue(2, 2)
    process(1, 1)

    def triple(k, carry):
        for j in range(3):
            i = 2 + k * 3 + j
            wait_scatter(j)
            issue(i + 1, j)
            process(i, (2 + j) % 3)
        return carry

    lax.fori_loop(0, loops, triple, 0)

    for i in range(2 + 3 * loops, nb):
        wait_scatter((i + 1) % 3)
        if i + 1 < nb:
            issue(i + 1, (i + 1) % 3)
        process(i, i % 3)
    wait_scatter((nb - 2) % 3)
    wait_scatter((nb - 1) % 3)
    plsc.subcore_barrier()

    @pl.when(sid < 2)
    def _():
        off = sid * _RPT_BIG
        pltpu.sync_copy(agg_sh.at[pl.ds(off, _RPT_BIG)],
                        out_hbm.at[cid, pl.ds(off, _RPT_BIG)])

    @pl.when(sid >= 2)
    def _():
        off = sid * _RPT_SMALL + 2 * (_RPT_BIG - _RPT_SMALL)
        pltpu.sync_copy(agg_sh.at[pl.ds(off, _RPT_SMALL)],
                        out_hbm.at[cid, pl.ds(off, _RPT_SMALL)])


def _sc_edge_call(nf, m, src, dst, zeros):
    n_edges = src.shape[0]
    epw = n_edges // _NW
    nb = epw // _B
    mesh = plsc.VectorSubcoreMesh(core_axis_name="c", subcore_axis_name="s")
    f = functools.partial(
        pl.kernel,
        out_type=jax.ShapeDtypeStruct((_NC, _N, _D_IN), jnp.float32),
        mesh=mesh,
        scratch_types=[
            pltpu.VMEM((epw,), jnp.int32),
            [pltpu.VMEM((_B,), jnp.int32) for _ in range(3)],
            [pltpu.VMEM((_B, _D_IN), jnp.float32) for _ in range(3)],
            [pltpu.VMEM((_B, _D_IN), jnp.float32) for _ in range(3)],
            pltpu.VMEM_SHARED((_N, _D_IN), jnp.float32),
            [pltpu.SemaphoreType.DMA for _ in range(3)],
            [pltpu.SemaphoreType.DMA for _ in range(3)],
            [pltpu.SemaphoreType.DMA for _ in range(3)],
            [pltpu.SemaphoreType.DMA for _ in range(3)],
        ],
    )(functools.partial(_sc_edge_body, epw))
    return f(nf, m, src.reshape(_NW, epw), dst, zeros)


def kernel(node_input, edge_src, edge_dst, edge_attr, edge_scalar_attr,
           W_self, W_fc1, W_fc2, W_tp, W_out):
    w_self = W_self / math.sqrt(_D_IN)
    w1t = (W_fc1 / math.sqrt(_D_SCALAR)).T
    w2t = (W_fc2 / math.sqrt(_FC0)).T
    wtpt = (jnp.transpose(W_tp, (1, 2, 0)).reshape(_D_IN, _D_EDGE * _FC1)
            / (math.sqrt(_FC1) * math.sqrt(_D_EDGE))).astype(jnp.bfloat16)
    wout = W_out * (math.sin(_MIX) / (math.sqrt(_NUM_NEIGHBORS)
                                      * math.sqrt(_D_IN)))

    nf, node_self_out = pl.pallas_call(
        _node_linear_body,
        grid=(_N // _NODE_BLK,),
        in_specs=[
            pl.BlockSpec((_NODE_BLK, _D_IN), lambda i: (i, 0)),
            pl.BlockSpec((_D_IN, 2 * _D_IN), lambda i: (0, 0)),
        ],
        out_specs=[
            pl.BlockSpec((_NODE_BLK, _D_IN), lambda i: (i, 0)),
            pl.BlockSpec((_NODE_BLK, _D_IN), lambda i: (i, 0)),
        ],
        out_shape=[
            jax.ShapeDtypeStruct((_N, _D_IN), jnp.float32),
            jax.ShapeDtypeStruct((_N, _D_IN), jnp.float32),
        ],
    )(node_input, w_self)

    esa_t = edge_scalar_attr.T
    ea_t = edge_attr.T
    half = _E // 2
    hblk = half // _EDGE_BLK

    def mlp_half(off):
        return pl.pallas_call(
            _edge_mlp_body,
            grid=(hblk,),
            in_specs=[
                pl.BlockSpec((_D_SCALAR, _EDGE_BLK), lambda i, o=off: (0, i + o)),
                pl.BlockSpec((_D_EDGE, _EDGE_BLK), lambda i, o=off: (0, i + o)),
                pl.BlockSpec((_FC0, _D_SCALAR), lambda i: (0, 0)),
                pl.BlockSpec((_FC1, _FC0), lambda i: (0, 0)),
                pl.BlockSpec((_D_IN, _D_EDGE * _FC1), lambda i: (0, 0)),
            ],
            out_specs=pl.BlockSpec((_EDGE_BLK, _D_IN), lambda i: (i, 0)),
            out_shape=jax.ShapeDtypeStruct((half, _D_IN), jnp.float32),
        )(esa_t, ea_t, w1t, w2t, wtpt)

    m0 = mlp_half(0)
    m1 = mlp_half(hblk)

    zeros = jnp.zeros((_N, _D_IN), jnp.float32)
    agg_a = _sc_edge_call(nf, m0, edge_src[:half], edge_dst[:half], zeros)
    agg_b = _sc_edge_call(nf, m1, edge_src[half:], edge_dst[half:], zeros)

    out = pl.pallas_call(
        _final_body,
        grid=(_N // _NODE_BLK,),
        in_specs=[
            pl.BlockSpec((_NC, _NODE_BLK, _D_IN), lambda i: (0, i, 0)),
            pl.BlockSpec((_NC, _NODE_BLK, _D_IN), lambda i: (0, i, 0)),
            pl.BlockSpec((_NODE_BLK, _D_IN), lambda i: (i, 0)),
            pl.BlockSpec((_D_IN, _D_IN), lambda i: (0, 0)),
        ],
        out_specs=pl.BlockSpec((_NODE_BLK, _D_IN), lambda i: (i, 0)),
        out_shape=jax.ShapeDtypeStruct((_N, _D_IN), jnp.float32),
    )(agg_a, agg_b, node_self_out, wout)
    return out


# speedup vs baseline: 10.6754x; 1.0077x over previous
import functools
import math

import jax
import jax.numpy as jnp
from jax import lax
from jax.experimental import pallas as pl
from jax.experimental.pallas import tpu as pltpu
from jax.experimental.pallas import tpu_sc as plsc

_N = 10000
_E = 320000
_D_IN = 128
_D_EDGE = 4
_D_SCALAR = 16
_FC0 = 64
_FC1 = 64
_NUM_NEIGHBORS = 32.0
_MIX = math.pi / 8.0

_NC = 2
_NS = 16
_NW = _NC * _NS
_B = 40
_EPW = _E // _NW
_NB = _EPW // _B
_RPT_BIG = 632
_RPT_SMALL = 624

_NODE_BLK = 2000
_EDGE_BLK = 3200


def _node_linear_body(x_ref, w_ref, nf_ref, so_ref):
    t = jnp.dot(x_ref[...], w_ref[...], preferred_element_type=jnp.float32)
    nf_ref[...] = t[:, :_D_IN]
    so_ref[...] = t[:, _D_IN:]


def _edge_mlp_body(esa_ref, ea_ref, w1t_ref, w2t_ref, wtpt_ref, m_ref):
    h = jax.nn.gelu(
        jnp.dot(w1t_ref[...], esa_ref[...], preferred_element_type=jnp.float32))
    h = jax.nn.gelu(
        jnp.dot(w2t_ref[...], h, preferred_element_type=jnp.float32))
    hb = h.astype(jnp.bfloat16)
    ea = ea_ref[...].astype(jnp.bfloat16)
    p = jnp.concatenate([hb * ea[v:v + 1, :] for v in range(_D_EDGE)], axis=0)
    mt = jnp.dot(wtpt_ref[...], p, preferred_element_type=jnp.float32)
    m_ref[...] = mt.T


def _final_body(agg_a_ref, agg_b_ref, so_ref, wout_ref, o_ref):
    a = (agg_a_ref[0] + agg_a_ref[1]) + (agg_b_ref[0] + agg_b_ref[1])
    o_ref[...] = so_ref[...] * math.cos(_MIX) + jnp.dot(
        a, wout_ref[...], preferred_element_type=jnp.float32)


def _sc_edge_body(epw, nf_hbm, m_hbm, src_hbm, dst_hbm, zeros_hbm, out_hbm,
                  src_all, dst_v, rows_v, m_v, agg_sh, gsem, msem, ssem,
                  dsem):
    nb = epw // _B
    cid = lax.axis_index("c")
    sid = lax.axis_index("s")
    wid = sid * _NC + cid
    base = wid * epw

    pltpu.sync_copy(src_hbm.at[wid], src_all)

    @pl.when(sid < 2)
    def _():
        off = sid * _RPT_BIG
        pltpu.sync_copy(zeros_hbm.at[pl.ds(off, _RPT_BIG)],
                        agg_sh.at[pl.ds(off, _RPT_BIG)])

    @pl.when(sid >= 2)
    def _():
        off = sid * _RPT_SMALL + 2 * (_RPT_BIG - _RPT_SMALL)
        pltpu.sync_copy(zeros_hbm.at[pl.ds(off, _RPT_SMALL)],
                        agg_sh.at[pl.ds(off, _RPT_SMALL)])

    plsc.subcore_barrier()

    def issue(i, slot):
        pltpu.async_copy(nf_hbm.at[src_all.at[pl.ds(i * _B, _B)]],
                         rows_v[slot], gsem[slot])
        pltpu.async_copy(m_hbm.at[pl.ds(base + i * _B, _B)], m_v[slot],
                         msem[slot])
        pltpu.async_copy(dst_hbm.at[pl.ds(base + i * _B, _B)], dst_v[slot],
                         dsem[slot])

    def process(i, slot):
        pltpu.make_async_copy(nf_hbm.at[pl.ds(0, _B)], rows_v[slot],
                              gsem[slot]).wait()
        pltpu.make_async_copy(m_hbm.at[pl.ds(base, _B)], m_v[slot],
                              msem[slot]).wait()

        @plsc.parallel_loop(0, _B, 1, unroll=4)
        def _mul(r):
            for c in range(_D_IN // 16):
                sl = pl.ds(c * 16, 16)
                rows_v[slot][r, sl] = rows_v[slot][r, sl] * m_v[slot][r, sl]

        pltpu.make_async_copy(dst_hbm.at[pl.ds(base, _B)], dst_v[slot],
                              dsem[slot]).wait()
        pltpu.async_copy(rows_v[slot], agg_sh.at[dst_v[slot]],
                         ssem[slot], add=True)

    def wait_scatter(slot):
        pltpu.make_async_copy(rows_v[slot], agg_sh.at[dst_v[slot]],
                              ssem[slot]).wait()

    tail = 2 + (nb - 4) % 3
    loops = (nb - 2 - tail) // 3
    issue(0, 0)
    issue(1, 1)
    process(0, 0)
    issue(2, 2)
    process(1, 1)

    def triple(k, carry):
        for j in range(3):
            i = 2 + k * 3 + j
            wait_scatter(j)
            issue(i + 1, j)
            process(i, (2 + j) % 3)
        return carry

    lax.fori_loop(0, loops, triple, 0)

    for i in range(2 + 3 * loops, nb):
        wait_scatter((i + 1) % 3)
        if i + 1 < nb:
            issue(i + 1, (i + 1) % 3)
        process(i, i % 3)
    wait_scatter((nb - 2) % 3)
    wait_scatter((nb - 1) % 3)
    plsc.subcore_barrier()

    @pl.when(sid < 2)
    def _():
        off = sid * _RPT_BIG
        pltpu.sync_copy(agg_sh.at[pl.ds(off, _RPT_BIG)],
                        out_hbm.at[cid, pl.ds(off, _RPT_BIG)])

    @pl.when(sid >= 2)
    def _():
        off = sid * _RPT_SMALL + 2 * (_RPT_BIG - _RPT_SMALL)
        pltpu.sync_copy(agg_sh.at[pl.ds(off, _RPT_SMALL)],
                        out_hbm.at[cid, pl.ds(off, _RPT_SMALL)])


def _sc_edge_call(nf, m, src, dst, zeros):
    n_edges = src.shape[0]
    epw = n_edges // _NW
    nb = epw // _B
    mesh = plsc.VectorSubcoreMesh(core_axis_name="c", subcore_axis_name="s")
    f = functools.partial(
        pl.kernel,
        out_type=jax.ShapeDtypeStruct((_NC, _N, _D_IN), jnp.float32),
        mesh=mesh,
        scratch_types=[
            pltpu.VMEM((epw,), jnp.int32),
            [pltpu.VMEM((_B,), jnp.int32) for _ in range(3)],
            [pltpu.VMEM((_B, _D_IN), jnp.float32) for _ in range(3)],
            [pltpu.VMEM((_B, _D_IN), jnp.float32) for _ in range(3)],
            pltpu.VMEM_SHARED((_N, _D_IN), jnp.float32),
            [pltpu.SemaphoreType.DMA for _ in range(3)],
            [pltpu.SemaphoreType.DMA for _ in range(3)],
            [pltpu.SemaphoreType.DMA for _ in range(3)],
            [pltpu.SemaphoreType.DMA for _ in range(3)],
        ],
    )(functools.partial(_sc_edge_body, epw))
    return f(nf, m, src.reshape(_NW, epw), dst, zeros)


def kernel(node_input, edge_src, edge_dst, edge_attr, edge_scalar_attr,
           W_self, W_fc1, W_fc2, W_tp, W_out):
    w_self = W_self / math.sqrt(_D_IN)
    w1t = (W_fc1 / math.sqrt(_D_SCALAR)).T
    w2t = (W_fc2 / math.sqrt(_FC0)).T
    wtpt = (jnp.transpose(W_tp, (1, 2, 0)).reshape(_D_IN, _D_EDGE * _FC1)
            / (math.sqrt(_FC1) * math.sqrt(_D_EDGE))).astype(jnp.bfloat16)
    wout = W_out * (math.sin(_MIX) / (math.sqrt(_NUM_NEIGHBORS)
                                      * math.sqrt(_D_IN)))

    nf, node_self_out = pl.pallas_call(
        _node_linear_body,
        grid=(_N // _NODE_BLK,),
        in_specs=[
            pl.BlockSpec((_NODE_BLK, _D_IN), lambda i: (i, 0)),
            pl.BlockSpec((_D_IN, 2 * _D_IN), lambda i: (0, 0)),
        ],
        out_specs=[
            pl.BlockSpec((_NODE_BLK, _D_IN), lambda i: (i, 0)),
            pl.BlockSpec((_NODE_BLK, _D_IN), lambda i: (i, 0)),
        ],
        out_shape=[
            jax.ShapeDtypeStruct((_N, _D_IN), jnp.float32),
            jax.ShapeDtypeStruct((_N, _D_IN), jnp.float32),
        ],
    )(node_input, w_self)

    esa_t = edge_scalar_attr.T
    ea_t = edge_attr.T
    half = _E // 2
    hblk = half // _EDGE_BLK

    def mlp_half(off):
        return pl.pallas_call(
            _edge_mlp_body,
            grid=(hblk,),
            in_specs=[
                pl.BlockSpec((_D_SCALAR, _EDGE_BLK), lambda i, o=off: (0, i + o)),
                pl.BlockSpec((_D_EDGE, _EDGE_BLK), lambda i, o=off: (0, i + o)),
                pl.BlockSpec((_FC0, _D_SCALAR), lambda i: (0, 0)),
                pl.BlockSpec((_FC1, _FC0), lambda i: (0, 0)),
                pl.BlockSpec((_D_IN, _D_EDGE * _FC1), lambda i: (0, 0)),
            ],
            out_specs=pl.BlockSpec((_EDGE_BLK, _D_IN), lambda i: (i, 0)),
            out_shape=jax.ShapeDtypeStruct((half, _D_IN), jnp.float32),
        )(esa_t, ea_t, w1t, w2t, wtpt)

    m0 = mlp_half(0)
    m1 = mlp_half(hblk)

    zeros = jnp.zeros((_N, _D_IN), jnp.float32)
    agg_a = _sc_edge_call(nf, m0, edge_src[:half], edge_dst[:half], zeros)
    agg_b = _sc_edge_call(nf, m1, edge_src[half:], edge_dst[half:], zeros)

    out = pl.pallas_call(
        _final_body,
        grid=(_N // _NODE_BLK,),
        in_specs=[
            pl.BlockSpec((_NC, _NODE_BLK, _D_IN), lambda i: (0, i, 0)),
            pl.BlockSpec((_NC, _NODE_BLK, _D_IN), lambda i: (0, i, 0)),
            pl.BlockSpec((_NODE_BLK, _D_IN), lambda i: (i, 0)),
            pl.BlockSpec((_D_IN, _D_IN), lambda i: (0, 0)),
        ],
        out_specs=pl.BlockSpec((_NODE_BLK, _D_IN), lambda i: (i, 0)),
        out_shape=jax.ShapeDtypeStruct((_N, _D_IN), jnp.float32),
    )(agg_a, agg_b, node_self_out, wout)
    return out
